# Initial kernel scaffold; baseline (speedup 1.0000x reference)
import jax
import jax.numpy as jnp
from jax.experimental import pallas as pl


def kernel(x, codebook):
    raise NotImplementedError("write your pallas kernel here")


# speedup vs baseline: 1.4865x; 1.4865x over previous
import functools

import jax
import jax.numpy as jnp
from jax import lax
from jax.experimental import pallas as pl
from jax.experimental.pallas import tpu as pltpu
from jax.experimental.pallas import tpu_sc as plsc

N, K, D = 65536, 1024, 32
R = 2048
NB = N // R


def _argmin_body(x_ref, cb_ref, out_ref):
    x = x_ref[...]
    cb = cb_ref[...]
    mm = lax.dot_general(x, cb, (((1,), (1,)), ((), ())),
                         preferred_element_type=jnp.float32)
    xsq = jnp.sum(x * x, axis=1, keepdims=True)
    csq = jnp.sum(cb * cb, axis=1)[None, :]
    dist2 = xsq + csq - 2.0 * mm
    out_ref[0, 0, :] = jnp.argmin(dist2, axis=1).astype(jnp.int32)


_argmin_call = pl.pallas_call(
    _argmin_body,
    grid=(NB,),
    in_specs=[
        pl.BlockSpec((R, D), lambda i: (i, 0)),
        pl.BlockSpec((K, D), lambda i: (0, 0)),
    ],
    out_specs=pl.BlockSpec((1, 1, R), lambda i: (i, 0, 0)),
    out_shape=jax.ShapeDtypeStruct((NB, 1, R), jnp.int32),
)

_NC, _NS = 2, 16
NW = _NC * _NS
BPW = N // NW


DP = 128
C = 512
NCHUNK = BPW // C


@functools.cache
def _make_sc_gather():
    @functools.partial(
        pl.kernel,
        mesh=plsc.VectorSubcoreMesh(core_axis_name="c", subcore_axis_name="s"),
        out_type=jax.ShapeDtypeStruct((N, DP), jnp.float32),
        scratch_types=[
            pltpu.VMEM((BPW,), jnp.int32),
            pltpu.VMEM((C, DP), jnp.float32),
            pltpu.SemaphoreType.DMA,
        ],
    )
    def _sc_gather(cb_hbm, idx_hbm, out_hbm, idx_v, rows_v, sem):
        wid = lax.axis_index("s") * _NC + lax.axis_index("c")
        base = wid * BPW
        pltpu.sync_copy(idx_hbm.at[pl.ds(base, BPW)], idx_v)

        def chunk(i, carry):
            pltpu.async_copy(cb_hbm.at[idx_v.at[pl.ds(i * C, C)]],
                             rows_v, sem).wait()
            pltpu.sync_copy(rows_v, out_hbm.at[pl.ds(base + i * C, C)])
            return carry

        lax.fori_loop(0, NCHUNK, chunk, 0)

    return _sc_gather


def kernel(x, codebook):
    indices = _argmin_call(x, codebook).reshape(N)
    cb_pad = jnp.zeros((K, DP), jnp.float32).at[:, :D].set(codebook)
    return _make_sc_gather()(cb_pad, indices)[:, :D]
